# baseline (speedup 1.0000x reference)
import functools

import jax
import jax.numpy as jnp
from jax import lax
from jax.experimental import pallas as pl
from jax.experimental.pallas import tpu as pltpu
from jax.experimental.pallas import tpu_sc as plsc

B = 4096
D = 128
NH = 50
NWI = 20
NT = 10

NC = 2
NS = 16
NWRK = NC * NS
IPW = B // NWRK
LANE = 16

HCI = 4
WCI = 8
TCI = 16
HROWS = HCI * NH
WROWS = WCI * NWI
TROWS = TCI * NT
BUF_ROWS = 200

_mesh = plsc.VectorSubcoreMesh(core_axis_name="c", subcore_axis_name="s")


def _wid():
    return lax.axis_index("s") * NC + lax.axis_index("c")


def _acc_rows(buf, row_base, n_rows, out_ref, out_row, scale, init=None):
    nlane = D // LANE

    def body(r, acc):
        return tuple(
            acc[l] + buf[row_base + r, pl.ds(l * LANE, LANE)]
            for l in range(nlane)
        )

    acc0 = tuple(jnp.zeros((LANE,), jnp.float32) for _ in range(nlane))
    acc = lax.fori_loop(0, n_rows, body, acc0)
    for l in range(nlane):
        sl = pl.ds(l * LANE, LANE)
        v = acc[l] * scale
        if init == "add":
            out_ref[out_row, sl] = out_ref[out_row, sl] + v
        else:
            out_ref[out_row, sl] = v


@functools.partial(
    pl.kernel,
    out_type=jax.ShapeDtypeStruct((B, D), jnp.float32),
    mesh=_mesh,
    scratch_types=[
        pltpu.VMEM((NH * IPW,), jnp.int32),
        pltpu.VMEM((NWI * IPW,), jnp.int32),
        pltpu.VMEM((IPW, D), jnp.float32),
        pltpu.VMEM((IPW, D), jnp.float32),
        pltpu.VMEM((IPW, D), jnp.float32),
        pltpu.VMEM((IPW, D), jnp.float32),
        pltpu.SemaphoreType.DMA,
        pltpu.SemaphoreType.DMA,
    ],
)
def _user_pool(hist_hbm, wish_hbm, book_hbm, x_hbm, hidx, widx, hacc0, hacc1,
               wacc0, wacc1, hsem, wsem):
    wid = _wid()
    pltpu.sync_copy(hist_hbm.at[wid], hidx)
    pltpu.sync_copy(wish_hbm.at[wid], widx)

    zeros = jnp.zeros((LANE,), jnp.float32)

    @pl.loop(0, IPW)
    def _(r):
        for l in range(D // LANE):
            sl = pl.ds(l * LANE, LANE)
            hacc0[r, sl] = zeros
            hacc1[r, sl] = zeros
            wacc0[r, sl] = zeros
            wacc1[r, sl] = zeros

    @pl.loop(0, NH, step=2)
    def _(k):
        pltpu.async_copy(book_hbm.at[hidx.at[pl.ds(k * IPW, IPW)]], hacc0,
                         hsem, add=True)
        pltpu.async_copy(book_hbm.at[hidx.at[pl.ds((k + 1) * IPW, IPW)]],
                         hacc1, hsem, add=True)

    @pl.loop(0, NWI, step=2)
    def _(k):
        pltpu.async_copy(book_hbm.at[widx.at[pl.ds(k * IPW, IPW)]], wacc0,
                         wsem, add=True)
        pltpu.async_copy(book_hbm.at[widx.at[pl.ds((k + 1) * IPW, IPW)]],
                         wacc1, wsem, add=True)

    @pl.loop(0, NH)
    def _(k):
        pltpu.make_async_copy(book_hbm.at[hidx.at[pl.ds(0, IPW)]], hacc0,
                              hsem).wait()

    @pl.loop(0, NWI)
    def _(k):
        pltpu.make_async_copy(book_hbm.at[widx.at[pl.ds(0, IPW)]], wacc0,
                              wsem).wait()

    @pl.loop(0, IPW)
    def _(r):
        for l in range(D // LANE):
            sl = pl.ds(l * LANE, LANE)
            hacc0[r, sl] = (hacc0[r, sl] + hacc1[r, sl]) * (1.0 / NH) + \
                (wacc0[r, sl] + wacc1[r, sl]) * (1.0 / NWI)

    pltpu.sync_copy(hacc0, x_hbm.at[pl.ds(wid * IPW, IPW)])


_T_OFF = 0
_B_OFF = IPW * NT
_A_OFF = _B_OFF + IPW
_L_OFF = _A_OFF + IPW
_ITM_W = _L_OFF + IPW


@functools.partial(
    pl.kernel,
    out_type=jax.ShapeDtypeStruct((B, D), jnp.float32),
    mesh=_mesh,
    scratch_types=[
        pltpu.VMEM((_ITM_W,), jnp.int32),
        pltpu.VMEM((IPW, D), jnp.float32),
        pltpu.VMEM((IPW, D), jnp.float32),
        pltpu.SemaphoreType.DMA,
        pltpu.SemaphoreType.DMA,
    ],
)
def _item_pool(itm_hbm, book_hbm, auth_hbm, lang_hbm, tag_hbm, i_hbm,
               iidx, iacc, tacc, semi, semt):
    wid = _wid()
    pltpu.sync_copy(itm_hbm.at[wid], iidx)

    zeros = jnp.zeros((LANE,), jnp.float32)

    @pl.loop(0, IPW)
    def _(r):
        for l in range(D // LANE):
            sl = pl.ds(l * LANE, LANE)
            iacc[r, sl] = zeros
            tacc[r, sl] = zeros

    pltpu.async_copy(book_hbm.at[iidx.at[pl.ds(_B_OFF, IPW)]], iacc, semi,
                     add=True)
    pltpu.async_copy(auth_hbm.at[iidx.at[pl.ds(_A_OFF, IPW)]], iacc, semi,
                     add=True)
    pltpu.async_copy(lang_hbm.at[iidx.at[pl.ds(_L_OFF, IPW)]], iacc, semi,
                     add=True)

    @pl.loop(0, NT)
    def _(k):
        pltpu.async_copy(tag_hbm.at[iidx.at[pl.ds(_T_OFF + k * IPW, IPW)]],
                         tacc, semt, add=True)

    for _ in range(3):
        pltpu.make_async_copy(book_hbm.at[iidx.at[pl.ds(_B_OFF, IPW)]],
                              iacc, semi).wait()

    @pl.loop(0, NT)
    def _(k):
        pltpu.make_async_copy(tag_hbm.at[iidx.at[pl.ds(_T_OFF, IPW)]],
                              tacc, semt).wait()

    @pl.loop(0, IPW)
    def _(r):
        for l in range(D // LANE):
            sl = pl.ds(l * LANE, LANE)
            iacc[r, sl] = iacc[r, sl] + tacc[r, sl] * (1.0 / NT)

    pltpu.sync_copy(iacc, i_hbm.at[pl.ds(wid * IPW, IPW)])


_BM = 1024


def _mlp_body(x_ref, w1, b1, w2, b2, w3, b3, w4, b4, o_ref):
    f32 = jnp.float32
    h = jnp.maximum(jnp.dot(x_ref[...], w1[...], preferred_element_type=f32)
                    + b1[...], 0.0)
    h = jnp.maximum(jnp.dot(h, w2[...], preferred_element_type=f32)
                    + b2[...], 0.0)
    h = jnp.maximum(jnp.dot(h, w3[...], preferred_element_type=f32)
                    + b3[...], 0.0)
    o_ref[...] = jnp.dot(h, w4[...], preferred_element_type=f32) + b4[...]


def _user_mlp(x, uW1, ub1, uW2, ub2, uW3, ub3, uW4, ub4):
    full = lambda s: pl.BlockSpec(s, lambda i: (0, 0))
    return pl.pallas_call(
        _mlp_body,
        grid=(B // _BM,),
        in_specs=[
            pl.BlockSpec((_BM, D), lambda i: (i, 0)),
            full(uW1.shape), full(ub1.shape),
            full(uW2.shape), full(ub2.shape),
            full(uW3.shape), full(ub3.shape),
            full(uW4.shape), full(ub4.shape),
        ],
        out_specs=pl.BlockSpec((_BM, D), lambda i: (i, 0)),
        out_shape=jax.ShapeDtypeStruct((B, D), jnp.float32),
        compiler_params=pltpu.CompilerParams(
            dimension_semantics=("arbitrary",)),
    )(x, uW1, ub1, uW2, ub2, uW3, ub3, uW4, ub4)


def _combine_body(u_ref, ip_ref, dn_ref, w1, b1, w2, b2, o_ref):
    f32 = jnp.float32
    h = jnp.maximum(jnp.dot(dn_ref[...], w1[...], preferred_element_type=f32)
                    + b1[...], 0.0)
    d = jnp.dot(h, w2[...], preferred_element_type=f32) + b2[...]
    o_ref[...] = jnp.sum(u_ref[...] * (ip_ref[...] + d), axis=1,
                         keepdims=True)


def _combine(u, ipart, dense8, dW1p, db1, dW2, db2):
    full = lambda s: pl.BlockSpec(s, lambda i: (0, 0))
    return pl.pallas_call(
        _combine_body,
        grid=(B // _BM,),
        in_specs=[
            pl.BlockSpec((_BM, D), lambda i: (i, 0)),
            pl.BlockSpec((_BM, D), lambda i: (i, 0)),
            pl.BlockSpec((_BM, 8), lambda i: (i, 0)),
            full(dW1p.shape), full(db1.shape),
            full(dW2.shape), full(db2.shape),
        ],
        out_specs=pl.BlockSpec((_BM, 1), lambda i: (i, 0)),
        out_shape=jax.ShapeDtypeStruct((B, 1), jnp.float32),
        compiler_params=pltpu.CompilerParams(
            dimension_semantics=("arbitrary",)),
    )(u, ipart, dense8, dW1p, db1, dW2, db2)


def kernel(hist_ids, wish_ids, bid, auth, lang, tags, dense, book_emb,
           auth_emb, lang_emb, tag_emb, dW1, db1, dW2, db2, uW1, ub1,
           uW2, ub2, uW3, ub3, uW4, ub4):
    i32 = jnp.int32
    hist_r = hist_ids.astype(i32).reshape(NWRK, IPW, NH).transpose(0, 2, 1) \
        .reshape(NWRK, NH * IPW)
    wish_r = wish_ids.astype(i32).reshape(NWRK, IPW, NWI).transpose(0, 2, 1) \
        .reshape(NWRK, NWI * IPW)
    tags_t = tags.astype(i32).reshape(NWRK, IPW, NT).transpose(0, 2, 1) \
        .reshape(NWRK, NT * IPW)
    itm = jnp.concatenate(
        [tags_t,
         bid.astype(i32).reshape(NWRK, IPW),
         auth.astype(i32).reshape(NWRK, IPW),
         lang.astype(i32).reshape(NWRK, IPW)],
        axis=1,
    )

    x = _user_pool(hist_r, wish_r, book_emb)
    ipart = _item_pool(itm, book_emb, auth_emb, lang_emb, tag_emb)

    u = _user_mlp(x, uW1, ub1.reshape(1, -1), uW2, ub2.reshape(1, -1),
                  uW3, ub3.reshape(1, -1), uW4, ub4.reshape(1, -1))

    dense8 = jnp.pad(dense, ((0, 0), (0, 5)))
    dW1p = jnp.pad(dW1, ((0, 5), (0, 0)))
    return _combine(u, ipart, dense8, dW1p, db1.reshape(1, -1), dW2,
                    db2.reshape(1, -1))
